# baseline (speedup 1.0000x reference)
import functools

import jax
import jax.numpy as jnp
from jax import lax
from jax.scipy.special import logsumexp
from jax.experimental import pallas as pl
from jax.experimental.pallas import tpu as pltpu
from jax.experimental.pallas import tpu_sc as plsc

_NUM_P = 65536
_U = 8
_DX = 128
_DU = 32
_ALPHA = 1.0
_EXP_FACTOR = 2.0
_NU = _NUM_P * _U

_NC = 2
_NS = 16
_NW = _NC * _NS
_ROWS_W = _NUM_P // _NW
_CHUNK = 128
_NCHUNK = _ROWS_W // _CHUNK


_sc_mesh = plsc.VectorSubcoreMesh(core_axis_name="c", subcore_axis_name="s")


def _sc_gather_body(particles_hbm, u4_hbm, g_hbm, q_hbm,
                    x_out, u4_out,
                    idx_g, idx_q, xbuf, ubuf, sem_x, sem_u):
    wid = lax.axis_index("s") * _NC + lax.axis_index("c")
    base0 = wid * _ROWS_W
    row0 = wid * _NCHUNK
    pltpu.sync_copy(g_hbm.at[pl.ds(row0, _NCHUNK)], idx_g)
    pltpu.sync_copy(q_hbm.at[pl.ds(row0, _NCHUNK)], idx_q)

    def body(ci, carry):
        base = base0 + ci * _CHUNK
        cx = pltpu.async_copy(particles_hbm.at[idx_g.at[ci]], xbuf, sem_x)
        cu = pltpu.async_copy(u4_hbm.at[idx_q.at[ci]], ubuf, sem_u)
        cx.wait()
        cu.wait()
        pltpu.sync_copy(xbuf, x_out.at[pl.ds(base, _CHUNK)])
        pltpu.sync_copy(ubuf, u4_out.at[pl.ds(base, _CHUNK)])
        return carry

    lax.fori_loop(0, _NCHUNK, body, 0)


def _make_sc_gather(interpret=False):
    return pl.kernel(
        _sc_gather_body,
        out_type=(
            jax.ShapeDtypeStruct((_NUM_P, _DX), jnp.float32),
            jax.ShapeDtypeStruct((_NUM_P, _DX), jnp.float32),
        ),
        mesh=_sc_mesh,
        interpret=interpret,
        scratch_types=[
            pltpu.VMEM((_NCHUNK, _CHUNK), jnp.int32),
            pltpu.VMEM((_NCHUNK, _CHUNK), jnp.int32),
            pltpu.VMEM((_CHUNK, _DX), jnp.float32),
            pltpu.VMEM((_CHUNK, _DX), jnp.float32),
            pltpu.SemaphoreType.DMA,
            pltpu.SemaphoreType.DMA,
        ],
    )


_sc_gather = _make_sc_gather()


_BR = 1024


def _dyn_body(x_ref, u4_ref, s_ref, nx_ref, a_ref, b_ref, np_ref, cat_ref):
    x = x_ref[...]
    u4 = u4_ref[...]
    k = (s_ref[0, 0, :] & 3).reshape(_BR, 1)
    u = jnp.where(
        k < 2,
        jnp.where(k == 0, u4[:, 0:_DU], u4[:, _DU:2 * _DU]),
        jnp.where(k == 2, u4[:, 2 * _DU:3 * _DU], u4[:, 3 * _DU:4 * _DU]),
    )
    np_ref[...] = (
        jnp.dot(x, a_ref[...], preferred_element_type=jnp.float32,
                precision=lax.Precision.HIGHEST)
        + jnp.dot(u, b_ref[...], preferred_element_type=jnp.float32,
                  precision=lax.Precision.HIGHEST)
        + nx_ref[...]
    )
    cat_ref[...] = jnp.concatenate([x, u], axis=1)


def _dynamics(x_sel, u4_sel, samples, noise_x, A, B):
    return pl.pallas_call(
        _dyn_body,
        grid=(_NUM_P // _BR,),
        in_specs=[
            pl.BlockSpec((_BR, _DX), lambda i: (i, 0)),
            pl.BlockSpec((_BR, _DX), lambda i: (i, 0)),
            pl.BlockSpec((1, 1, _BR), lambda i: (i, 0, 0)),
            pl.BlockSpec((_BR, _DX), lambda i: (i, 0)),
            pl.BlockSpec((_DX, _DX), lambda i: (0, 0)),
            pl.BlockSpec((_DU, _DX), lambda i: (0, 0)),
        ],
        out_specs=[
            pl.BlockSpec((_BR, _DX), lambda i: (i, 0)),
            pl.BlockSpec((_BR, _DX + _DU), lambda i: (i, 0)),
        ],
        out_shape=[
            jax.ShapeDtypeStruct((_NUM_P, _DX), jnp.float32),
            jax.ShapeDtypeStruct((_NUM_P, _DX + _DU), jnp.float32),
        ],
    )(x_sel, u4_sel, samples.reshape(_NUM_P // _BR, 1, _BR), noise_x, A, B)


def kernel(particles, noise_u, resample_offsets, noise_x, K, log_sig, Q, R, A, B, iteration):
    mu = particles @ K
    mu_rep = jnp.repeat(mu, _U, axis=0)
    sig = jnp.exp(log_sig)
    new_u = mu_rep + sig * noise_u
    qx = 0.5 * jnp.sum((particles @ Q) * particles, axis=1)
    cost = jnp.repeat(qx, _U) + 0.5 * jnp.sum((new_u @ R) * new_u, axis=1)
    u_corr = (-_EXP_FACTOR ** 2 + 1) / (2.0 * _EXP_FACTOR ** 2) * jnp.sum(((new_u - mu_rep) / sig) ** 2, axis=1)
    log_weights = -_ALPHA * cost + jnp.log(_EXP_FACTOR) + u_corr
    log_norm = log_weights - logsumexp(log_weights)
    cdf = jnp.cumsum(jnp.exp(log_norm))
    positions = (jnp.arange(_NUM_P, dtype=jnp.float32) + resample_offsets) / _NUM_P
    samples = jnp.clip(jnp.searchsorted(cdf, positions), 0, _NU - 1)
    samples_div = samples // _U

    samples32 = samples.astype(jnp.int32)
    x_sel, u4_sel = _sc_gather(
        particles,
        new_u.reshape(_NU // 4, 4 * _DU),
        samples_div.astype(jnp.int32).reshape(_NUM_P // _CHUNK, _CHUNK),
        (samples32 >> 2).reshape(_NUM_P // _CHUNK, _CHUNK),
    )
    lw_sel = log_weights[samples]

    new_particles, particles_cat = _dynamics(x_sel, u4_sel, samples32, noise_x, A, B)

    return (new_particles, particles_cat, lw_sel, samples_div)


# speedup vs baseline: 1.8574x; 1.8574x over previous
import functools

import jax
import jax.numpy as jnp
from jax import lax
from jax.scipy.special import logsumexp
from jax.experimental import pallas as pl
from jax.experimental.pallas import tpu as pltpu
from jax.experimental.pallas import tpu_sc as plsc

_NUM_P = 65536
_U = 8
_DX = 128
_DU = 32
_ALPHA = 1.0
_EXP_FACTOR = 2.0
_NU = _NUM_P * _U

_NC = 2
_NS = 16
_NW = _NC * _NS
_ROWS_W = _NUM_P // _NW
_CHUNK = 128
_NCHUNK = _ROWS_W // _CHUNK


_sc_mesh = plsc.VectorSubcoreMesh(core_axis_name="c", subcore_axis_name="s")


_NBLK = _NU // _DX
_NMID = _NU // 16
_L1_STEPS = (2048, 1024, 512, 256, 128, 64, 32, 16, 8, 4, 2, 1)


def _sc_resample_body(pos_hbm, coarse_hbm, mid_hbm, cdfblk_hbm,
                      particles_hbm, u4_hbm, lw4_hbm,
                      samp_out, sdiv_out, x_out, u4_out, lwsel_out,
                      coarsebuf, midbuf, posc, blkidx, gidx, qidx, ridx,
                      sampbuf, sdivbuf, lwselbuf,
                      blkbuf, xbuf, ubuf, lwbuf,
                      sem_b, sem_x, sem_u, sem_l):
    wid = lax.axis_index("s") * _NC + lax.axis_index("c")
    row0 = wid * _NCHUNK
    base0 = wid * _ROWS_W
    pltpu.sync_copy(coarse_hbm, coarsebuf)
    pltpu.sync_copy(mid_hbm, midbuf)
    iota16 = jnp.arange(16, dtype=jnp.int32)

    def body(ci, carry):
        pltpu.sync_copy(pos_hbm.at[row0 + ci], posc)
        pvals, bfins, scnts, sbars = [], [], [], []
        for v in range(8):
            p = posc[pl.ds(v * 16, 16)]
            b = jnp.zeros((16,), jnp.int32)
            for step in _L1_STEPS:
                cv = plsc.load_gather(coarsebuf, [b + (step - 1)])
                b = jnp.where(cv < p, b + step, b)
            cv = plsc.load_gather(coarsebuf, [b])
            bfin = jnp.where(cv < p, b + 1, b)
            bbar = jnp.minimum(bfin, _NBLK - 1)
            s = jnp.zeros((16,), jnp.int32)
            for k in range(8):
                mv = plsc.load_gather(midbuf, [bbar * 8 + k])
                s = s + jnp.where(mv < p, 1, 0)
            blkidx[pl.ds(v * 16, 16)] = bbar
            pvals.append(p)
            bfins.append(bfin)
            scnts.append(s)
            sbars.append(jnp.minimum(s, 7))
        pltpu.async_copy(cdfblk_hbm.at[blkidx], blkbuf, sem_b).wait()
        for v in range(8):
            p, bfin, s, sbar = pvals[v], bfins[v], scnts[v], sbars[v]
            q_ids = v * 16 + iota16
            colbase = sbar * 16
            c16 = jnp.zeros((16,), jnp.int32)
            for k in range(16):
                vv = plsc.load_gather(blkbuf, [q_ids, colbase + k])
                c16 = c16 + jnp.where(vv < p, 1, 0)
            raw = bfin * _DX + s * 16 + c16
            samp = jnp.minimum(raw, _NU - 1)
            sl = pl.ds(v * 16, 16)
            sampbuf[sl] = samp
            sdivbuf[sl] = lax.shift_right_logical(samp, 3)
            gidx[sl] = lax.shift_right_logical(samp, 3)
            qidx[sl] = lax.shift_right_logical(samp, 2)
            ridx[sl] = lax.shift_right_logical(samp, 7)
            pvals[v] = samp & 127
        cx = pltpu.async_copy(particles_hbm.at[gidx], xbuf, sem_x)
        cu = pltpu.async_copy(u4_hbm.at[qidx], ubuf, sem_u)
        cl = pltpu.async_copy(lw4_hbm.at[ridx], lwbuf, sem_l)
        cx.wait()
        cu.wait()
        cl.wait()
        for v in range(8):
            lvals = plsc.load_gather(lwbuf, [v * 16 + iota16, pvals[v]])
            lwselbuf[pl.ds(v * 16, 16)] = lvals
        base = base0 + ci * _CHUNK
        pltpu.sync_copy(sampbuf, samp_out.at[row0 + ci])
        pltpu.sync_copy(sdivbuf, sdiv_out.at[row0 + ci])
        pltpu.sync_copy(lwselbuf, lwsel_out.at[row0 + ci])
        pltpu.sync_copy(xbuf, x_out.at[pl.ds(base, _CHUNK)])
        pltpu.sync_copy(ubuf, u4_out.at[pl.ds(base, _CHUNK)])
        return carry

    lax.fori_loop(0, _NCHUNK, body, 0)


def _make_sc_resample(interpret=False):
    nrow = _NUM_P // _CHUNK
    return pl.kernel(
        _sc_resample_body,
        out_type=(
            jax.ShapeDtypeStruct((nrow, _CHUNK), jnp.int32),
            jax.ShapeDtypeStruct((nrow, _CHUNK), jnp.int32),
            jax.ShapeDtypeStruct((_NUM_P, _DX), jnp.float32),
            jax.ShapeDtypeStruct((_NUM_P, _DX), jnp.float32),
            jax.ShapeDtypeStruct((nrow, _CHUNK), jnp.float32),
        ),
        mesh=_sc_mesh,
        compiler_params=pltpu.CompilerParams(needs_layout_passes=False),
        interpret=interpret,
        scratch_types=[
            pltpu.VMEM((_NBLK,), jnp.float32),
            pltpu.VMEM((_NMID,), jnp.float32),
            pltpu.VMEM((_CHUNK,), jnp.float32),
            pltpu.VMEM((_CHUNK,), jnp.int32),
            pltpu.VMEM((_CHUNK,), jnp.int32),
            pltpu.VMEM((_CHUNK,), jnp.int32),
            pltpu.VMEM((_CHUNK,), jnp.int32),
            pltpu.VMEM((_CHUNK,), jnp.int32),
            pltpu.VMEM((_CHUNK,), jnp.int32),
            pltpu.VMEM((_CHUNK,), jnp.float32),
            pltpu.VMEM((_CHUNK, _DX), jnp.float32),
            pltpu.VMEM((_CHUNK, _DX), jnp.float32),
            pltpu.VMEM((_CHUNK, _DX), jnp.float32),
            pltpu.VMEM((_CHUNK, _DX), jnp.float32),
            pltpu.SemaphoreType.DMA,
            pltpu.SemaphoreType.DMA,
            pltpu.SemaphoreType.DMA,
            pltpu.SemaphoreType.DMA,
        ],
    )


_sc_resample = _make_sc_resample()


_BR = 1024


def _dyn_body(x_ref, u4_ref, s_ref, nx_ref, a_ref, b_ref, np_ref, cat_ref):
    x = x_ref[...]
    u4 = u4_ref[...]
    k = (s_ref[0, 0, :] & 3).reshape(_BR, 1)
    u = jnp.where(
        k < 2,
        jnp.where(k == 0, u4[:, 0:_DU], u4[:, _DU:2 * _DU]),
        jnp.where(k == 2, u4[:, 2 * _DU:3 * _DU], u4[:, 3 * _DU:4 * _DU]),
    )
    np_ref[...] = (
        jnp.dot(x, a_ref[...], preferred_element_type=jnp.float32,
                precision=lax.Precision.HIGHEST)
        + jnp.dot(u, b_ref[...], preferred_element_type=jnp.float32,
                  precision=lax.Precision.HIGHEST)
        + nx_ref[...]
    )
    cat_ref[...] = jnp.concatenate([x, u], axis=1)


def _dynamics(x_sel, u4_sel, samples, noise_x, A, B):
    return pl.pallas_call(
        _dyn_body,
        grid=(_NUM_P // _BR,),
        in_specs=[
            pl.BlockSpec((_BR, _DX), lambda i: (i, 0)),
            pl.BlockSpec((_BR, _DX), lambda i: (i, 0)),
            pl.BlockSpec((1, 1, _BR), lambda i: (i, 0, 0)),
            pl.BlockSpec((_BR, _DX), lambda i: (i, 0)),
            pl.BlockSpec((_DX, _DX), lambda i: (0, 0)),
            pl.BlockSpec((_DU, _DX), lambda i: (0, 0)),
        ],
        out_specs=[
            pl.BlockSpec((_BR, _DX), lambda i: (i, 0)),
            pl.BlockSpec((_BR, _DX + _DU), lambda i: (i, 0)),
        ],
        out_shape=[
            jax.ShapeDtypeStruct((_NUM_P, _DX), jnp.float32),
            jax.ShapeDtypeStruct((_NUM_P, _DX + _DU), jnp.float32),
        ],
    )(x_sel, u4_sel, samples.reshape(_NUM_P // _BR, 1, _BR), noise_x, A, B)


def kernel(particles, noise_u, resample_offsets, noise_x, K, log_sig, Q, R, A, B, iteration):
    mu = particles @ K
    mu_rep = jnp.repeat(mu, _U, axis=0)
    sig = jnp.exp(log_sig)
    new_u = mu_rep + sig * noise_u
    qx = 0.5 * jnp.sum((particles @ Q) * particles, axis=1)
    cost = jnp.repeat(qx, _U) + 0.5 * jnp.sum((new_u @ R) * new_u, axis=1)
    u_corr = (-_EXP_FACTOR ** 2 + 1) / (2.0 * _EXP_FACTOR ** 2) * jnp.sum(((new_u - mu_rep) / sig) ** 2, axis=1)
    log_weights = -_ALPHA * cost + jnp.log(_EXP_FACTOR) + u_corr
    log_norm = log_weights - logsumexp(log_weights)
    cdf = jnp.cumsum(jnp.exp(log_norm))
    positions = (jnp.arange(_NUM_P, dtype=jnp.float32) + resample_offsets) / _NUM_P

    samp2, sdiv2, x_sel, u4_sel, lwsel2 = _sc_resample(
        positions.reshape(_NUM_P // _CHUNK, _CHUNK),
        cdf[_DX - 1::_DX],
        cdf[15::16],
        cdf.reshape(_NBLK, _DX),
        particles,
        new_u.reshape(_NU // 4, 4 * _DU),
        log_weights.reshape(_NBLK, _DX),
    )
    samples32 = samp2.reshape(_NUM_P)
    samples_div = sdiv2.reshape(_NUM_P)
    lw_sel = lwsel2.reshape(_NUM_P)

    new_particles, particles_cat = _dynamics(x_sel, u4_sel, samples32, noise_x, A, B)

    return (new_particles, particles_cat, lw_sel, samples_div)
